# baseline (speedup 1.0000x reference)
import functools

import jax
import jax.numpy as jnp
from jax import lax
from jax.experimental import pallas as pl
from jax.experimental.pallas import tpu as pltpu
from jax.experimental.pallas import tpu_sc as plsc

N = 10000
E = 320000
D = 128
G = 64
OUT = 16

NC = 2
NS = 16
L = 16
NW = NC * NS
EPW = E // NW
RPT = 640
PADROW = RPT
ACCR = RPT + 8
KB = 128
CD = 2000

_MESH = dict(core_axis_name="c", subcore_axis_name="s")
_SC_PARAMS = pltpu.CompilerParams(needs_layout_passes=False)


def _proj_body(x_ref, w1_ref, b1_ref, a_ref, b_ref):
    wt = w1_ref[0:D, :]
    wb = w1_ref[D:2 * D, :]
    if x_ref.shape == (NC, N, D):
        xv = jnp.maximum(x_ref[0], x_ref[1])
    else:
        xv = x_ref[...]
    xb = xv.astype(jnp.bfloat16)
    a_ref[...] = (jnp.dot(xb, (wt - wb).astype(jnp.bfloat16),
                          preferred_element_type=jnp.float32) + b1_ref[...])
    b_ref[...] = jnp.dot(xb, wb.astype(jnp.bfloat16),
                         preferred_element_type=jnp.float32)


def _node_proj(x, W1, b1):
    return pl.pallas_call(
        _proj_body,
        out_shape=(jax.ShapeDtypeStruct((N, D), jnp.float32),
                   jax.ShapeDtypeStruct((N, D), jnp.float32)),
    )(x, W1, b1.reshape(1, D))


def _mm_body(h_ref, w_ref, b_ref, o_ref):
    hb = h_ref[...].astype(jnp.bfloat16)
    o_ref[...] = (jnp.dot(hb, w_ref[...].astype(jnp.bfloat16),
                          preferred_element_type=jnp.float32) + b_ref[...])


def _edge_mm(h, W2, b2):
    BM = 2560
    return pl.pallas_call(
        _mm_body,
        grid=(E // BM,),
        in_specs=[pl.BlockSpec((BM, D), lambda i: (i, 0)),
                  pl.BlockSpec((D, D), lambda i: (0, 0)),
                  pl.BlockSpec((1, D), lambda i: (0, 0))],
        out_specs=pl.BlockSpec((BM, D), lambda i: (i, 0)),
        out_shape=jax.ShapeDtypeStruct((E, D), jnp.float32),
    )(h, W2, b2.reshape(1, D))


def _head_body(h_ref, batch_ref, wl_ref, bl_ref, wl2_ref, bl2_ref,
               wo_ref, bo_ref, o_ref):
    hv = jnp.maximum(h_ref[0], h_ref[1])
    b2d = batch_ref[...]
    gid = lax.broadcasted_iota(jnp.int32, (G, N), 0)
    oh = (b2d == gid).astype(jnp.float32)
    s = jnp.dot(oh, hv, preferred_element_type=jnp.float32)
    cnt = jnp.sum(oh, axis=1, keepdims=True)
    inv = 1.0 / jnp.maximum(cnt, 1.0)
    z = jnp.maximum(jnp.dot(s * inv, wl_ref[...],
                            preferred_element_type=jnp.float32)
                    + bl_ref[...], 0.0)
    z = jnp.maximum(jnp.dot(z, wl2_ref[...],
                            preferred_element_type=jnp.float32)
                    + bl2_ref[...], 0.0)
    z = jnp.maximum(jnp.dot(z, wo_ref[...],
                            preferred_element_type=jnp.float32)
                    + bo_ref[...], 0.0)
    zmax = jnp.max(z, axis=0, keepdims=True)
    ez = jnp.exp(z - zmax)
    o_ref[...] = ez / jnp.sum(ez, axis=0, keepdims=True)


def _head(h, batch, Wl, bl, Wl2, bl2, Wo, bo):
    return pl.pallas_call(
        _head_body,
        out_shape=jax.ShapeDtypeStruct((G, OUT), jnp.float32),
    )(h, batch.reshape(1, N), Wl, bl.reshape(1, -1),
      Wl2, bl2.reshape(1, -1), Wo, bo.reshape(1, -1))


CG = 128
NCHG = EPW // CG
TAILG = EPW - NCHG * CG


def _gather_body(a_hbm, b_hbm, dst_hbm, src_hbm, h_hbm,
                 dsti, srci, ar0, br0, ar1, br1,
                 sa0, sb0, sa1, sb1, ss0, ss1):
    wid = lax.axis_index("s") * NC + lax.axis_index("c")
    base = pl.multiple_of(wid * EPW, 8)
    pltpu.sync_copy(dst_hbm.at[pl.ds(base, EPW)], dsti)
    pltpu.sync_copy(src_hbm.at[pl.ds(base, EPW)], srci)
    ar = (ar0, ar1)
    br = (br0, br1)
    sa = (sa0, sa1)
    sb = (sb0, sb1)
    ss = (ss0, ss1)

    def issue(t, p):
        o = pl.multiple_of(t * CG, 8)
        pltpu.async_copy(a_hbm.at[dsti.at[pl.ds(o, CG)]], ar[p], sa[p])
        pltpu.async_copy(b_hbm.at[srci.at[pl.ds(o, CG)]], br[p], sb[p])

    zf = jnp.zeros((L,), jnp.float32)

    def compute(p, nrows):
        def edge(i, c2):
            for j in range(D // L):
                sl = pl.ds(j * L, L)
                ar[p][i, sl] = jnp.maximum(ar[p][i, sl] + br[p][i, sl], zf)
            return c2

        lax.fori_loop(0, nrows, edge, 0, unroll=2)

    issue(0, 0)

    def pair(pp, carry):
        for par in (0, 1):
            t = 2 * pp + par
            pltpu.make_async_copy(a_hbm.at[dsti.at[pl.ds(0, CG)]],
                                  ar[par], sa[par]).wait()
            pltpu.make_async_copy(b_hbm.at[srci.at[pl.ds(0, CG)]],
                                  br[par], sb[par]).wait()

            @pl.when(t + 1 < NCHG)
            def _():
                @pl.when(t >= 1)
                def _():
                    pltpu.make_async_copy(
                        ar[1 - par],
                        h_hbm.at[pl.ds(base, CG)], ss[1 - par]).wait()

                issue(t + 1, 1 - par)

            compute(par, CG)
            pltpu.async_copy(
                ar[par],
                h_hbm.at[pl.ds(pl.multiple_of(base + t * CG, 8), CG)],
                ss[par])
        return carry

    lax.fori_loop(0, NCHG // 2, pair, 0)
    pltpu.make_async_copy(ar0, h_hbm.at[pl.ds(base, CG)], ss0).wait()
    to = pl.multiple_of(base + NCHG * CG, 8)
    pltpu.async_copy(a_hbm.at[dsti.at[pl.ds(NCHG * CG, TAILG)]],
                     ar0.at[pl.ds(0, TAILG)], sa0).wait()
    pltpu.async_copy(b_hbm.at[srci.at[pl.ds(NCHG * CG, TAILG)]],
                     br0.at[pl.ds(0, TAILG)], sb0).wait()
    compute(0, TAILG)
    pltpu.sync_copy(ar0.at[pl.ds(0, TAILG)], h_hbm.at[pl.ds(to, TAILG)])
    pltpu.make_async_copy(ar1, h_hbm.at[pl.ds(base, CG)], ss1).wait()


def _edge_gather(A, B, dst, src):
    mesh = plsc.VectorSubcoreMesh(**_MESH)
    f = pl.kernel(
        _gather_body,
        out_type=jax.ShapeDtypeStruct((E, D), jnp.float32),
        mesh=mesh,
        compiler_params=_SC_PARAMS,
        scratch_types=[
            pltpu.VMEM((EPW,), jnp.int32),
            pltpu.VMEM((EPW,), jnp.int32),
            pltpu.VMEM((CG, D), jnp.float32),
            pltpu.VMEM((CG, D), jnp.float32),
            pltpu.VMEM((CG, D), jnp.float32),
            pltpu.VMEM((CG, D), jnp.float32),
            pltpu.SemaphoreType.DMA,
            pltpu.SemaphoreType.DMA,
            pltpu.SemaphoreType.DMA,
            pltpu.SemaphoreType.DMA,
            pltpu.SemaphoreType.DMA,
            pltpu.SemaphoreType.DMA,
        ],
    )
    return f(A, B, dst, src)


def _scatter_body(m_hbm, dst_hbm, out_hbm, acc, db0, db1, selp, idxb, rows,
                  sd, sb0, sb1):
    c = lax.axis_index("c")
    s = lax.axis_index("s")
    lo = s * RPT
    ebase = c * (E // NC)
    iota = lax.iota(jnp.int32, L)
    zf = jnp.zeros((L,), jnp.float32)
    padp = jnp.full((L,), PADROW, jnp.int32)

    def zr(r, cc):
        for j in range(D // L):
            acc[r, pl.ds(j * L, L)] = zf
        return cc

    lax.fori_loop(0, ACCR, zr, 0)

    def initsel(v, cc):
        selp[pl.ds(v * L, L)] = padp
        return cc

    lax.fori_loop(0, (KB + L) // L, initsel, 0)

    def fire(cursor):
        def up(j, cc):
            sl = pl.ds(j * L, L)
            idxb[sl] = lax.shift_right_logical(selp[sl], 10)
            return cc

        lax.fori_loop(0, KB // L, up, 0)
        pltpu.async_copy(m_hbm.at[idxb], rows, sd).wait()

        def proc(i, cc):
            r = selp[pl.ds(i, L)][0] & 1023
            for j in range(D // L):
                sl = pl.ds(j * L, L)
                acc[r, sl] = jnp.maximum(acc[r, sl], rows[i, sl])
            return cc

        lax.fori_loop(0, KB, proc, 0)
        selp[pl.ds(0, L)] = selp[pl.ds(KB, L)]
        return cursor - KB

    def scan_chunk(db, off, cursor):
        def vec(v, cur):
            d = db[pl.ds(v * L, L)]
            msk = (d >= lo) & (d < lo + RPT)
            cnt = plsc.all_reduce_population_count(msk)[0]

            def sel(cur2):
                packed = lax.shift_left(off + v * L + iota, 10) | (d - lo)
                plsc.store_compressed(selp.at[pl.ds(cur2, L)], packed,
                                      mask=msk)
                return lax.cond(cur2 + cnt >= KB, fire,
                                lambda x: x, cur2 + cnt)

            return lax.cond(cnt > 0, sel, lambda x: x, cur)

        return lax.fori_loop(0, CD // L, vec, cursor)

    NCH = (E // NC) // CD
    cp0 = pltpu.async_copy(dst_hbm.at[pl.ds(ebase, CD)], db0, sb0)

    def pair(p, cursor):
        off0 = pl.multiple_of(ebase + (2 * p) * CD, 8)
        cp0 = pltpu.make_async_copy(dst_hbm.at[pl.ds(off0, CD)], db0, sb0)
        cp0.wait()
        off1 = pl.multiple_of(off0 + CD, 8)
        pltpu.async_copy(dst_hbm.at[pl.ds(off1, CD)], db1, sb1)
        cursor = scan_chunk(db0, off0, cursor)
        pltpu.make_async_copy(dst_hbm.at[pl.ds(off1, CD)], db1, sb1).wait()

        @pl.when(p < NCH // 2 - 1)
        def _():
            off2 = pl.multiple_of(off1 + CD, 8)
            pltpu.async_copy(dst_hbm.at[pl.ds(off2, CD)], db0, sb0)

        return scan_chunk(db1, off1, cursor)

    cursor = lax.fori_loop(0, NCH // 2, pair, 0)

    def padtail(v, cc):
        base = v * L
        msk = (base + iota) >= cursor
        selp[pl.ds(base, L)] = jnp.where(msk, padp, selp[pl.ds(base, L)])
        return cc

    lax.fori_loop(0, KB // L, padtail, 0)
    fire(0)

    rem = N - (NS - 1) * RPT

    @pl.when(s < NS - 1)
    def _():
        pltpu.sync_copy(acc.at[pl.ds(0, RPT)], out_hbm.at[c, pl.ds(lo, RPT)])

    @pl.when(s == NS - 1)
    def _():
        pltpu.sync_copy(acc.at[pl.ds(0, rem)], out_hbm.at[c, pl.ds(lo, rem)])


def _seg_max(m, dst):
    mesh = plsc.VectorSubcoreMesh(**_MESH)
    f = pl.kernel(
        _scatter_body,
        out_type=jax.ShapeDtypeStruct((NC, N, D), jnp.float32),
        mesh=mesh,
        compiler_params=_SC_PARAMS,
        scratch_types=[
            pltpu.VMEM((ACCR, D), jnp.float32),
            pltpu.VMEM((CD,), jnp.int32),
            pltpu.VMEM((CD,), jnp.int32),
            pltpu.VMEM((KB + L,), jnp.int32),
            pltpu.VMEM((KB,), jnp.int32),
            pltpu.VMEM((KB, D), jnp.float32),
            pltpu.SemaphoreType.DMA,
            pltpu.SemaphoreType.DMA,
            pltpu.SemaphoreType.DMA,
        ],
    )
    return f(m, dst)


def _conv(x, dst, src, W1, b1, W2, b2):
    A, B = _node_proj(x, W1, b1)
    h = _edge_gather(A, B, dst, src)
    m = _edge_mm(h, W2, b2)
    return _seg_max(m, dst)


def kernel(x, edge_index, batch, W1a, b1a, W2a, b2a, W1b, b1b, W2b, b2b,
           Wl, bl, Wl2, bl2, Wo, bo):
    src = edge_index[0]
    dst = edge_index[1]
    h1 = _conv(x, dst, src, W1a, b1a, W2a, b2a)
    h2 = _conv(h1, dst, src, W1b, b1b, W2b, b2b)
    return _head(h2, batch, Wl, bl, Wl2, bl2, Wo, bo)


# speedup vs baseline: 1.1710x; 1.1710x over previous
import functools

import jax
import jax.numpy as jnp
from jax import lax
from jax.experimental import pallas as pl
from jax.experimental.pallas import tpu as pltpu
from jax.experimental.pallas import tpu_sc as plsc

N = 10000
E = 320000
D = 128
G = 64
OUT = 16

NC = 2
NS = 16
L = 16
NW = NC * NS
EPW = E // NW
RPT = 640
PADROW = RPT
ACCR = RPT + 8
KB = 128
CD = 2000

_MESH = dict(core_axis_name="c", subcore_axis_name="s")
_SC_PARAMS = pltpu.CompilerParams(needs_layout_passes=False)


def _proj_body(x_ref, w1_ref, b1_ref, a_ref, b_ref):
    wt = w1_ref[0:D, :]
    wb = w1_ref[D:2 * D, :]
    if x_ref.shape == (NC, N, D):
        xv = jnp.maximum(x_ref[0], x_ref[1])
    else:
        xv = x_ref[...]
    xb = xv.astype(jnp.bfloat16)
    a_ref[...] = (jnp.dot(xb, (wt - wb).astype(jnp.bfloat16),
                          preferred_element_type=jnp.float32) + b1_ref[...])
    b_ref[...] = jnp.dot(xb, wb.astype(jnp.bfloat16),
                         preferred_element_type=jnp.float32)


def _node_proj(x, W1, b1):
    return pl.pallas_call(
        _proj_body,
        out_shape=(jax.ShapeDtypeStruct((N, D), jnp.float32),
                   jax.ShapeDtypeStruct((N, D), jnp.float32)),
    )(x, W1, b1.reshape(1, D))


def _mm_body(h_ref, w_ref, b_ref, o_ref):
    hb = h_ref[...].astype(jnp.bfloat16)
    o_ref[...] = (jnp.dot(hb, w_ref[...].astype(jnp.bfloat16),
                          preferred_element_type=jnp.float32) + b_ref[...])


def _edge_mm(h, W2, b2):
    BM = 2560
    return pl.pallas_call(
        _mm_body,
        grid=(E // BM,),
        in_specs=[pl.BlockSpec((BM, D), lambda i: (i, 0)),
                  pl.BlockSpec((D, D), lambda i: (0, 0)),
                  pl.BlockSpec((1, D), lambda i: (0, 0))],
        out_specs=pl.BlockSpec((BM, D), lambda i: (i, 0)),
        out_shape=jax.ShapeDtypeStruct((E, D), jnp.float32),
    )(h, W2, b2.reshape(1, D))


def _head_body(h_ref, batch_ref, wl_ref, bl_ref, wl2_ref, bl2_ref,
               wo_ref, bo_ref, o_ref):
    hv = jnp.maximum(h_ref[0], h_ref[1])
    b2d = batch_ref[...]
    gid = lax.broadcasted_iota(jnp.int32, (G, N), 0)
    oh = (b2d == gid).astype(jnp.float32)
    s = jnp.dot(oh, hv, preferred_element_type=jnp.float32)
    cnt = jnp.sum(oh, axis=1, keepdims=True)
    inv = 1.0 / jnp.maximum(cnt, 1.0)
    z = jnp.maximum(jnp.dot(s * inv, wl_ref[...],
                            preferred_element_type=jnp.float32)
                    + bl_ref[...], 0.0)
    z = jnp.maximum(jnp.dot(z, wl2_ref[...],
                            preferred_element_type=jnp.float32)
                    + bl2_ref[...], 0.0)
    z = jnp.maximum(jnp.dot(z, wo_ref[...],
                            preferred_element_type=jnp.float32)
                    + bo_ref[...], 0.0)
    zmax = jnp.max(z, axis=0, keepdims=True)
    ez = jnp.exp(z - zmax)
    o_ref[...] = ez / jnp.sum(ez, axis=0, keepdims=True)


def _head(h, batch, Wl, bl, Wl2, bl2, Wo, bo):
    return pl.pallas_call(
        _head_body,
        out_shape=jax.ShapeDtypeStruct((G, OUT), jnp.float32),
    )(h, batch.reshape(1, N), Wl, bl.reshape(1, -1),
      Wl2, bl2.reshape(1, -1), Wo, bo.reshape(1, -1))


CG = 128
NCHG = EPW // CG
TAILG = EPW - NCHG * CG


def _gather_body(a_hbm, b_hbm, dst_hbm, src_hbm, h_hbm,
                 dsti, srci, ar0, br0, ar1, br1,
                 sa0, sb0, sa1, sb1, ss0, ss1):
    wid = lax.axis_index("s") * NC + lax.axis_index("c")
    base = pl.multiple_of(wid * EPW, 8)
    pltpu.sync_copy(dst_hbm.at[pl.ds(base, EPW)], dsti)
    pltpu.sync_copy(src_hbm.at[pl.ds(base, EPW)], srci)
    ar = (ar0, ar1)
    br = (br0, br1)
    sa = (sa0, sa1)
    sb = (sb0, sb1)
    ss = (ss0, ss1)

    def issue(t, p):
        o = pl.multiple_of(t * CG, 8)
        pltpu.async_copy(a_hbm.at[dsti.at[pl.ds(o, CG)]], ar[p], sa[p])
        pltpu.async_copy(b_hbm.at[srci.at[pl.ds(o, CG)]], br[p], sb[p])

    zf = jnp.zeros((L,), jnp.float32)

    def compute(p, nrows):
        def edge(i, c2):
            for j in range(D // L):
                sl = pl.ds(j * L, L)
                ar[p][i, sl] = jnp.maximum(ar[p][i, sl] + br[p][i, sl], zf)
            return c2

        lax.fori_loop(0, nrows, edge, 0, unroll=4)

    issue(0, 0)

    def pair(pp, carry):
        for par in (0, 1):
            t = 2 * pp + par
            pltpu.make_async_copy(a_hbm.at[dsti.at[pl.ds(0, CG)]],
                                  ar[par], sa[par]).wait()
            pltpu.make_async_copy(b_hbm.at[srci.at[pl.ds(0, CG)]],
                                  br[par], sb[par]).wait()

            @pl.when(t + 1 < NCHG)
            def _():
                @pl.when(t >= 1)
                def _():
                    pltpu.make_async_copy(
                        ar[1 - par],
                        h_hbm.at[pl.ds(base, CG)], ss[1 - par]).wait()

                issue(t + 1, 1 - par)

            compute(par, CG)
            pltpu.async_copy(
                ar[par],
                h_hbm.at[pl.ds(pl.multiple_of(base + t * CG, 8), CG)],
                ss[par])
        return carry

    lax.fori_loop(0, NCHG // 2, pair, 0)
    pltpu.make_async_copy(ar0, h_hbm.at[pl.ds(base, CG)], ss0).wait()
    to = pl.multiple_of(base + NCHG * CG, 8)
    pltpu.async_copy(a_hbm.at[dsti.at[pl.ds(NCHG * CG, TAILG)]],
                     ar0.at[pl.ds(0, TAILG)], sa0).wait()
    pltpu.async_copy(b_hbm.at[srci.at[pl.ds(NCHG * CG, TAILG)]],
                     br0.at[pl.ds(0, TAILG)], sb0).wait()
    compute(0, TAILG)
    pltpu.sync_copy(ar0.at[pl.ds(0, TAILG)], h_hbm.at[pl.ds(to, TAILG)])
    pltpu.make_async_copy(ar1, h_hbm.at[pl.ds(base, CG)], ss1).wait()


def _edge_gather(A, B, dst, src):
    mesh = plsc.VectorSubcoreMesh(**_MESH)
    f = pl.kernel(
        _gather_body,
        out_type=jax.ShapeDtypeStruct((E, D), jnp.float32),
        mesh=mesh,
        compiler_params=_SC_PARAMS,
        scratch_types=[
            pltpu.VMEM((EPW,), jnp.int32),
            pltpu.VMEM((EPW,), jnp.int32),
            pltpu.VMEM((CG, D), jnp.float32),
            pltpu.VMEM((CG, D), jnp.float32),
            pltpu.VMEM((CG, D), jnp.float32),
            pltpu.VMEM((CG, D), jnp.float32),
            pltpu.SemaphoreType.DMA,
            pltpu.SemaphoreType.DMA,
            pltpu.SemaphoreType.DMA,
            pltpu.SemaphoreType.DMA,
            pltpu.SemaphoreType.DMA,
            pltpu.SemaphoreType.DMA,
        ],
    )
    return f(A, B, dst, src)


def _build_body(dst_hbm, lists_hbm, counts_hbm,
                db0, db1, selp, sout0, sout1, cntb,
                sb0, sb1, sw0, sw1):
    c = lax.axis_index("c")
    s = lax.axis_index("s")
    lo = s * RPT
    ebase = c * (E // NC)
    iota = lax.iota(jnp.int32, L)
    padp = jnp.full((L,), PADROW, jnp.int32)

    def fire(w):
        nf = lax.shift_right_logical(w, 16)

        def emit(sb, sem):
            @pl.when(nf >= 2)
            def _():
                pltpu.make_async_copy(
                    sb, lists_hbm.at[c, s, pl.ds(0, KB)], sem).wait()

            def cp(j, cc):
                sl = pl.ds(j * L, L)
                sb[sl] = selp[sl]
                return cc

            lax.fori_loop(0, KB // L, cp, 0, unroll=4)
            pltpu.async_copy(
                sb, lists_hbm.at[c, s,
                                 pl.ds(pl.multiple_of(nf * KB, 8), KB)], sem)
            return 0

        lax.cond((nf & 1) == 0,
                 lambda _: emit(sout0, sw0),
                 lambda _: emit(sout1, sw1), 0)
        selp[pl.ds(0, L)] = selp[pl.ds(KB, L)]
        return w - KB + 65536

    def scan_chunk(db, off, cursor):
        def vec(v, cur):
            d = db[pl.ds(v * L, L)]
            msk = (d >= lo) & (d < lo + RPT)
            cnt = plsc.all_reduce_population_count(msk)[0]

            def sel(w1):
                cur2 = w1 & 65535
                packed = lax.shift_left(off + v * L + iota, 10) | (d - lo)
                plsc.store_compressed(selp.at[pl.ds(cur2, L)], packed,
                                      mask=msk)
                return lax.cond(((w1 + cnt) & 65535) >= KB, fire,
                                lambda x: x, w1 + cnt)

            return lax.cond(cnt > 0, sel, lambda x: x, cur)

        return lax.fori_loop(0, CD // L, vec, cursor, unroll=4)

    NCH = (E // NC) // CD
    cp0 = pltpu.async_copy(dst_hbm.at[pl.ds(ebase, CD)], db0, sb0)

    def pair(p, cursor):
        off0 = pl.multiple_of(ebase + (2 * p) * CD, 8)
        cp0 = pltpu.make_async_copy(dst_hbm.at[pl.ds(off0, CD)], db0, sb0)
        cp0.wait()
        off1 = pl.multiple_of(off0 + CD, 8)
        pltpu.async_copy(dst_hbm.at[pl.ds(off1, CD)], db1, sb1)
        cursor = scan_chunk(db0, off0, cursor)
        pltpu.make_async_copy(dst_hbm.at[pl.ds(off1, CD)], db1, sb1).wait()

        @pl.when(p < NCH // 2 - 1)
        def _():
            off2 = pl.multiple_of(off1 + CD, 8)
            pltpu.async_copy(dst_hbm.at[pl.ds(off2, CD)], db0, sb0)

        return scan_chunk(db1, off1, cursor)

    w = lax.fori_loop(0, NCH // 2, pair, 0)
    cursor = w & 65535

    def padtail(v, cc):
        base = v * L
        msk = (base + iota) >= cursor
        selp[pl.ds(base, L)] = jnp.where(msk, padp, selp[pl.ds(base, L)])
        return cc

    lax.fori_loop(0, KB // L, padtail, 0)
    wf = fire(lax.shift_left(lax.shift_right_logical(w, 16), 16) + KB)
    nb = lax.shift_right_logical(wf, 16)

    cntb[pl.ds(0, L)] = jnp.full((L,), nb, jnp.int32)
    pltpu.sync_copy(cntb, counts_hbm.at[c, s])
    pltpu.make_async_copy(sout0, lists_hbm.at[c, s, pl.ds(0, KB)], sw0).wait()

    @pl.when(nb >= 2)
    def _():
        pltpu.make_async_copy(sout1, lists_hbm.at[c, s, pl.ds(0, KB)],
                              sw1).wait()


LCAP = E // NC + KB


def _build_lists(dst):
    mesh = plsc.VectorSubcoreMesh(**_MESH)
    f = pl.kernel(
        _build_body,
        out_type=(jax.ShapeDtypeStruct((NC, NS, LCAP), jnp.int32),
                  jax.ShapeDtypeStruct((NC, NS, L), jnp.int32)),
        mesh=mesh,
        compiler_params=_SC_PARAMS,
        scratch_types=[
            pltpu.VMEM((CD,), jnp.int32),
            pltpu.VMEM((CD,), jnp.int32),
            pltpu.VMEM((KB + L,), jnp.int32),
            pltpu.VMEM((KB,), jnp.int32),
            pltpu.VMEM((KB,), jnp.int32),
            pltpu.VMEM((L,), jnp.int32),
            pltpu.SemaphoreType.DMA,
            pltpu.SemaphoreType.DMA,
            pltpu.SemaphoreType.DMA,
            pltpu.SemaphoreType.DMA,
        ],
    )
    return f(dst)


def _scatter2_body(m_hbm, lists_hbm, counts_hbm, out_hbm,
                   acc, sel0, sel1, idxb, rows, cntb, sd, sl0, sl1):
    c = lax.axis_index("c")
    s = lax.axis_index("s")
    lo = s * RPT
    zf = jnp.zeros((L,), jnp.float32)

    def zr(r, cc):
        for j in range(D // L):
            acc[r, pl.ds(j * L, L)] = zf
        return cc

    lax.fori_loop(0, ACCR, zr, 0)

    pltpu.sync_copy(counts_hbm.at[c, s], cntb)
    nb = cntb[pl.ds(0, L)][0]
    pltpu.async_copy(lists_hbm.at[c, s, pl.ds(0, KB)],
                     sel0.at[pl.ds(0, KB)], sl0)

    def bat(b, cc):
        def go(cs, csem, ns, nsem):
            pltpu.make_async_copy(lists_hbm.at[c, s, pl.ds(0, KB)],
                                  cs.at[pl.ds(0, KB)], csem).wait()

            @pl.when(b + 1 < nb)
            def _():
                pltpu.async_copy(
                    lists_hbm.at[c, s,
                                 pl.ds(pl.multiple_of((b + 1) * KB, 8), KB)],
                    ns.at[pl.ds(0, KB)], nsem)

            def up(j, cc2):
                sl = pl.ds(j * L, L)
                idxb[sl] = lax.shift_right_logical(cs[sl], 10)
                return cc2

            lax.fori_loop(0, KB // L, up, 0, unroll=4)
            pltpu.async_copy(m_hbm.at[idxb], rows, sd).wait()

            def proc(i, cc2):
                r = cs[pl.ds(i, L)][0] & 1023
                for j in range(D // L):
                    sl = pl.ds(j * L, L)
                    acc[r, sl] = jnp.maximum(acc[r, sl], rows[i, sl])
                return cc2

            lax.fori_loop(0, KB, proc, 0, unroll=8)
            return 0

        return lax.cond((b & 1) == 0,
                        lambda _: go(sel0, sl0, sel1, sl1),
                        lambda _: go(sel1, sl1, sel0, sl0), 0)

    lax.fori_loop(0, nb, bat, 0)

    rem = N - (NS - 1) * RPT

    @pl.when(s < NS - 1)
    def _():
        pltpu.sync_copy(acc.at[pl.ds(0, RPT)], out_hbm.at[c, pl.ds(lo, RPT)])

    @pl.when(s == NS - 1)
    def _():
        pltpu.sync_copy(acc.at[pl.ds(0, rem)], out_hbm.at[c, pl.ds(lo, rem)])


def _seg_max_replay(m, lists, counts):
    mesh = plsc.VectorSubcoreMesh(**_MESH)
    f = pl.kernel(
        _scatter2_body,
        out_type=jax.ShapeDtypeStruct((NC, N, D), jnp.float32),
        mesh=mesh,
        compiler_params=_SC_PARAMS,
        scratch_types=[
            pltpu.VMEM((ACCR, D), jnp.float32),
            pltpu.VMEM((KB + L,), jnp.int32),
            pltpu.VMEM((KB + L,), jnp.int32),
            pltpu.VMEM((KB,), jnp.int32),
            pltpu.VMEM((KB, D), jnp.float32),
            pltpu.VMEM((L,), jnp.int32),
            pltpu.SemaphoreType.DMA,
            pltpu.SemaphoreType.DMA,
            pltpu.SemaphoreType.DMA,
        ],
    )
    return f(m, lists, counts)


def kernel(x, edge_index, batch, W1a, b1a, W2a, b2a, W1b, b1b, W2b, b2b,
           Wl, bl, Wl2, bl2, Wo, bo):
    src = edge_index[0]
    dst = edge_index[1]
    lists, counts = _build_lists(dst)
    A, B = _node_proj(x, W1a, b1a)
    h = _edge_gather(A, B, dst, src)
    m = _edge_mm(h, W2a, b2a)
    h1 = _seg_max_replay(m, lists, counts)
    A, B = _node_proj(h1, W1b, b1b)
    h = _edge_gather(A, B, dst, src)
    m = _edge_mm(h, W2b, b2b)
    h2 = _seg_max_replay(m, lists, counts)
    return _head(h2, batch, Wl, bl, Wl2, bl2, Wo, bo)
